# Initial kernel scaffold; baseline (speedup 1.0000x reference)
import jax
import jax.numpy as jnp
from jax.experimental import pallas as pl


def kernel(x, edge_index, drug_pos_ids, target_pos_ids, drug_neg_ids, target_neg_ids, adjacency_matrix, W_gcn0, b_gcn0, W_gcn1, b_gcn1, W_gcn2, b_gcn2, W_Q, W_K, W_V, ln_g, ln_b, W_out, b_out):
    raise NotImplementedError("write your pallas kernel here")


# speedup vs baseline: 16.7751x; 16.7751x over previous
import functools

import jax
import jax.numpy as jnp
from jax import lax
from jax.experimental import pallas as pl
from jax.experimental.pallas import tpu as pltpu
from jax.experimental.pallas import tpu_sc as plsc

N = 10000
NP = 10240
E = 320000
D = 128
HEADS = 8
HD = 16
B = 4096
EPS = 1e-5

NC = 2
NS = 16
NW = NC * NS
CHUNK = 128
NCHUNK = 79
EPAD = NW * NCHUNK * CHUNK - E
RPS = NP // NS

IPW = 4 * B // NW
ICH = IPW // CHUNK

_HIGH = lax.Precision.HIGHEST


def _dotT(a, w):
    return lax.dot_general(a, w, (((1,), (1,)), ((), ())),
                           precision=_HIGH, preferred_element_type=jnp.float32)


def _ln(t, g, b):
    m = jnp.mean(t, axis=-1, keepdims=True)
    v = jnp.mean((t - m) ** 2, axis=-1, keepdims=True)
    return (t - m) * lax.rsqrt(v + EPS) * g + b


DW = 128


@functools.cache
def _make_deg_kernel():
    return functools.partial(
        pl.kernel,
        out_type=jax.ShapeDtypeStruct((NC, NP, DW), jnp.float32),
        mesh=plsc.VectorSubcoreMesh(core_axis_name="core",
                                    subcore_axis_name="subcore"),
        scratch_types=[
            pltpu.VMEM((NCHUNK, CHUNK), jnp.int32),
            pltpu.VMEM((CHUNK, DW), jnp.float32),
            pltpu.VMEM_SHARED((NP, DW), jnp.float32),
        ],
    )(_deg_body)


def _deg_body(dst_hbm, zeros_hbm, ones_hbm, deg_out, idx_v, ones_v, deg_sh):
    core = lax.axis_index("core")
    sid = lax.axis_index("subcore")
    wid = core * NS + sid
    pltpu.sync_copy(ones_hbm, ones_v)
    pltpu.sync_copy(zeros_hbm, deg_sh.at[pl.ds(sid * RPS, RPS)])
    pltpu.sync_copy(dst_hbm.at[wid], idx_v)
    plsc.subcore_barrier()

    @pl.loop(0, NCHUNK)
    def _(j):
        pltpu.sync_copy(ones_v, deg_sh.at[idx_v.at[j]], add=True)

    plsc.subcore_barrier()
    pltpu.sync_copy(deg_sh.at[pl.ds(sid * RPS, RPS)],
                    deg_out.at[core, pl.ds(sid * RPS, RPS)])


@functools.cache
def _make_edge_kernel():
    return functools.partial(
        pl.kernel,
        out_type=jax.ShapeDtypeStruct((NC, NP, D), jnp.float32),
        mesh=plsc.VectorSubcoreMesh(core_axis_name="core",
                                    subcore_axis_name="subcore"),
        scratch_types=[
            pltpu.VMEM((NCHUNK, CHUNK), jnp.int32),
            pltpu.VMEM((NCHUNK, CHUNK), jnp.int32),
            pltpu.VMEM((CHUNK, D), jnp.float32),
            pltpu.VMEM_SHARED((NP, D), jnp.float32),
        ],
    )(_edge_body)


def _edge_body(src_hbm, dst_hbm, table_hbm, zeros_hbm, acc_out,
               sidx, didx, rows, acc_sh):
    core = lax.axis_index("core")
    sid = lax.axis_index("subcore")
    wid = core * NS + sid
    pltpu.sync_copy(zeros_hbm, acc_sh.at[pl.ds(sid * RPS, RPS)])
    pltpu.sync_copy(src_hbm.at[wid], sidx)
    pltpu.sync_copy(dst_hbm.at[wid], didx)
    plsc.subcore_barrier()

    @pl.loop(0, NCHUNK)
    def _(j):
        pltpu.sync_copy(table_hbm.at[sidx.at[j]], rows)
        pltpu.sync_copy(rows, acc_sh.at[didx.at[j]], add=True)

    plsc.subcore_barrier()
    pltpu.sync_copy(acc_sh.at[pl.ds(sid * RPS, RPS)],
                    acc_out.at[core, pl.ds(sid * RPS, RPS)])


@functools.cache
def _make_gather_kernel():
    return functools.partial(
        pl.kernel,
        out_type=jax.ShapeDtypeStruct((4 * B, D), jnp.float32),
        mesh=plsc.VectorSubcoreMesh(core_axis_name="core",
                                    subcore_axis_name="subcore"),
        scratch_types=[
            pltpu.VMEM((ICH, CHUNK), jnp.int32),
            pltpu.VMEM((CHUNK, D), jnp.float32),
        ],
    )(_gather_body)


def _gather_body(ids_hbm, zd_hbm, out_hbm, idx_v, rows_v):
    core = lax.axis_index("core")
    sid = lax.axis_index("subcore")
    wid = core * NS + sid
    pltpu.sync_copy(ids_hbm.at[wid], idx_v)

    @pl.loop(0, ICH)
    def _(j):
        pltpu.sync_copy(zd_hbm.at[idx_v.at[j]], rows_v)
        pltpu.sync_copy(rows_v, out_hbm.at[pl.ds(wid * IPW + j * CHUNK, CHUNK)])


_R = 1024
_G = NP // _R

_row_spec = pl.BlockSpec((_R, D), lambda i: (i, 0))
_col_spec = pl.BlockSpec((_R, 1), lambda i: (i, 0))
_acc_spec = pl.BlockSpec((NC, _R, D), lambda i: (0, i, 0))
_w_spec = pl.BlockSpec((D, D), lambda i: (0, 0))
_v_spec = pl.BlockSpec((1, D), lambda i: (0, 0))


def _pre_body(x_ref, deg_ref, dinv_ref, curs_ref):
    d = deg_ref[0, :, 0:1] + deg_ref[1, :, 0:1] + 1.0
    dv = lax.rsqrt(d)
    dinv_ref[...] = dv
    curs_ref[...] = x_ref[...] * dv


def _pre(xp, degp):
    return pl.pallas_call(
        _pre_body,
        grid=(_G,),
        in_specs=[_row_spec, pl.BlockSpec((NC, _R, DW), lambda i: (0, i, 0))],
        out_specs=[_col_spec, _row_spec],
        out_shape=[jax.ShapeDtypeStruct((NP, 1), jnp.float32),
                   jax.ShapeDtypeStruct((NP, D), jnp.float32)],
    )(xp, degp)


def _layer_body(acc_ref, cur_ref, dinv_ref, w_ref, b_ref, g_ref, bb_ref,
                cur_out, curs_out):
    dinv = dinv_ref[...]
    cur = cur_ref[...]
    agg = (acc_ref[0] + acc_ref[1] + cur * dinv) * dinv
    h = _dotT(agg, w_ref[...]) + b_ref[...]
    new = _ln(jnp.maximum(h, 0.0), g_ref[...], bb_ref[...])
    nxt = cur + new
    cur_out[...] = nxt
    curs_out[...] = nxt * dinv


def _layer(acc, cur, dinv, w, b, g, bb):
    return pl.pallas_call(
        _layer_body,
        grid=(_G,),
        in_specs=[_acc_spec, _row_spec, _col_spec, _w_spec, _v_spec, _v_spec,
                  _v_spec],
        out_specs=[_row_spec, _row_spec],
        out_shape=[jax.ShapeDtypeStruct((NP, D), jnp.float32),
                   jax.ShapeDtypeStruct((NP, D), jnp.float32)],
    )(acc, cur, dinv, w, b, g, bb)


def _final_body(acc_ref, e0_ref, e1_ref, dinv_ref, w2_ref, b2_ref,
                g_ref, bb_ref, wq_ref, wk_ref, wv_ref, wo_ref, bo_ref,
                zd_ref):
    dinv = dinv_ref[...]
    e1 = e1_ref[...]
    g = g_ref[...]
    bb = bb_ref[...]
    agg = (acc_ref[0] + acc_ref[1] + e1 * dinv) * dinv
    h = _dotT(agg, w2_ref[...]) + b2_ref[...]
    new = _ln(jnp.maximum(h, 0.0), g, bb)
    e2 = e1 + new
    e0 = e0_ref[...]

    q = _dotT(e2, wq_ref[...])
    ri = lax.broadcasted_iota(jnp.int32, (D, D), 0) // HD
    ci = lax.broadcasted_iota(jnp.int32, (D, D), 1) // HD
    rmat = jnp.where(ri == ci, 1.0, 0.0).astype(jnp.float32)

    def srep(e):
        k = _dotT(e, wk_ref[...])
        v = _dotT(e, wv_ref[...])
        s = lax.dot_general(q * k, rmat, (((1,), (0,)), ((), ())),
                            precision=_HIGH,
                            preferred_element_type=jnp.float32) * 0.25
        return s, v

    s0, v0 = srep(e0)
    s1, v1 = srep(e1)
    s2, v2 = srep(e2)
    m = jnp.maximum(jnp.maximum(s0, s1), s2)
    a0 = jnp.exp(s0 - m)
    a1 = jnp.exp(s1 - m)
    a2 = jnp.exp(s2 - m)
    ctx = (a0 * v0 + a1 * v1 + a2 * v2) / (a0 + a1 + a2)
    z = _ln(ctx, g, bb)
    zd_ref[...] = _dotT(z, wo_ref[...]) + bo_ref[...]


def _final(acc, e0, e1, dinv, w2, b2, g, bb, wq, wk, wv, wo, bo):
    return pl.pallas_call(
        _final_body,
        grid=(_G,),
        in_specs=[_acc_spec, _row_spec, _row_spec, _col_spec, _w_spec, _v_spec,
                  _v_spec, _v_spec, _w_spec, _w_spec, _w_spec, _w_spec,
                  _v_spec],
        out_specs=[_row_spec],
        out_shape=[jax.ShapeDtypeStruct((NP, D), jnp.float32)],
    )(acc, e0, e1, dinv, w2, b2, g, bb, wq, wk, wv, wo, bo)[0]


def kernel(x, edge_index, drug_pos_ids, target_pos_ids, drug_neg_ids,
           target_neg_ids, adjacency_matrix, W_gcn0, b_gcn0, W_gcn1, b_gcn1,
           W_gcn2, b_gcn2, W_Q, W_K, W_V, ln_g, ln_b, W_out, b_out):
    xp = jnp.pad(x, ((0, NP - N), (0, 0)))
    pad_idx = N + (jnp.arange(EPAD, dtype=jnp.int32) % (NP - N))
    src3 = jnp.concatenate([edge_index[0], pad_idx]).reshape(NW, NCHUNK, CHUNK)
    dst3 = jnp.concatenate([edge_index[1], pad_idx]).reshape(NW, NCHUNK, CHUNK)
    zeros_deg = jnp.zeros((RPS, DW), jnp.float32)
    zeros_rows = jnp.zeros((RPS, D), jnp.float32)
    ones_col = jnp.ones((CHUNK, DW), jnp.float32)

    b0 = b_gcn0.reshape(1, D)
    b1 = b_gcn1.reshape(1, D)
    b2 = b_gcn2.reshape(1, D)
    g = ln_g.reshape(1, D)
    bb = ln_b.reshape(1, D)
    bo = b_out.reshape(1, D)

    _deg_kernel = _make_deg_kernel()
    _edge_kernel = _make_edge_kernel()
    _gather_kernel = _make_gather_kernel()

    degp = _deg_kernel(dst3, zeros_deg, ones_col)
    dinv, curs0 = _pre(xp, degp)
    acc0 = _edge_kernel(src3, dst3, curs0, zeros_rows)
    cur1, curs1 = _layer(acc0, xp, dinv, W_gcn0, b0, g, bb)
    acc1 = _edge_kernel(src3, dst3, curs1, zeros_rows)
    cur2, curs2 = _layer(acc1, cur1, dinv, W_gcn1, b1, g, bb)
    acc2 = _edge_kernel(src3, dst3, curs2, zeros_rows)
    zd = _final(acc2, cur1, cur2, dinv, W_gcn2, b2, g, bb, W_Q, W_K, W_V,
                W_out, bo)

    ids = jnp.concatenate([drug_pos_ids, target_pos_ids, drug_neg_ids,
                           target_neg_ids]).reshape(NW, ICH, CHUNK)
    outg = _gather_kernel(ids, zd)
    return (outg[0:B], outg[B:2 * B], outg[2 * B:3 * B], outg[3 * B:4 * B])


# speedup vs baseline: 22.3893x; 1.3347x over previous
import functools

import jax
import jax.numpy as jnp
from jax import lax
from jax.experimental import pallas as pl
from jax.experimental.pallas import tpu as pltpu
from jax.experimental.pallas import tpu_sc as plsc

N = 10000
NP = 10240
E = 320000
D = 128
HEADS = 8
HD = 16
B = 4096
EPS = 1e-5

NC = 2
NS = 16
NW = NC * NS
CHUNK = 128
NCHUNK = 80
HC = NCHUNK // 2
EPAD = NW * NCHUNK * CHUNK - E
RPS = NP // NS

IPW = 4 * B // NW
ICH = IPW // CHUNK

_HIGH = lax.Precision.HIGHEST


def _dotT(a, w):
    return lax.dot_general(a, w, (((1,), (1,)), ((), ())),
                           precision=_HIGH, preferred_element_type=jnp.float32)


def _ln(t, g, b):
    m = jnp.mean(t, axis=-1, keepdims=True)
    v = jnp.mean((t - m) ** 2, axis=-1, keepdims=True)
    return (t - m) * lax.rsqrt(v + EPS) * g + b


DW = 128


@functools.cache
def _make_deg_kernel():
    return functools.partial(
        pl.kernel,
        out_type=jax.ShapeDtypeStruct((NC, NP, DW), jnp.float32),
        mesh=plsc.VectorSubcoreMesh(core_axis_name="core",
                                    subcore_axis_name="subcore"),
        scratch_types=[
            pltpu.VMEM((NCHUNK, CHUNK), jnp.int32),
            pltpu.VMEM((CHUNK, DW), jnp.float32),
            pltpu.VMEM_SHARED((NP, DW), jnp.float32),
            pltpu.SemaphoreType.DMA,
        ],
    )(_deg_body)


def _deg_body(dst_hbm, zeros_hbm, ones_hbm, deg_out, idx_v, ones_v, deg_sh,
              sem):
    core = lax.axis_index("core")
    sid = lax.axis_index("subcore")
    wid = core * NS + sid
    pltpu.sync_copy(ones_hbm, ones_v)
    pltpu.sync_copy(zeros_hbm, deg_sh.at[pl.ds(sid * RPS, RPS)])
    pltpu.sync_copy(dst_hbm.at[wid], idx_v)
    plsc.subcore_barrier()

    @pl.loop(0, NCHUNK)
    def _(j):
        pltpu.async_copy(ones_v, deg_sh.at[idx_v.at[j]], sem, add=True)

    @pl.loop(0, NCHUNK)
    def _(j):
        pltpu.make_async_copy(ones_v, deg_sh.at[idx_v.at[j]], sem).wait()

    plsc.subcore_barrier()
    pltpu.sync_copy(deg_sh.at[pl.ds(sid * RPS, RPS)],
                    deg_out.at[core, pl.ds(sid * RPS, RPS)])


@functools.cache
def _make_edge_kernel():
    return functools.partial(
        pl.kernel,
        out_type=jax.ShapeDtypeStruct((NC, NP, D), jnp.float32),
        mesh=plsc.VectorSubcoreMesh(core_axis_name="core",
                                    subcore_axis_name="subcore"),
        scratch_types=[
            pltpu.VMEM((HC, CHUNK), jnp.int32),
            pltpu.VMEM((HC, CHUNK), jnp.int32),
            pltpu.VMEM((2, CHUNK, D), jnp.float32),
            pltpu.VMEM_SHARED((NP, D), jnp.float32),
            pltpu.SemaphoreType.DMA,
            pltpu.SemaphoreType.DMA,
        ],
    )(_edge_body)


def _edge_body(src_hbm, dst_hbm, table_hbm, zeros_hbm, acc_out,
               sidx, didx, rows, acc_sh, g0, g1):
    core = lax.axis_index("core")
    sid = lax.axis_index("subcore")
    wid = core * NS + sid
    pltpu.sync_copy(zeros_hbm, acc_sh.at[pl.ds(sid * RPS, RPS)])
    plsc.subcore_barrier()

    @pl.loop(0, 2)
    def _(p):
        pltpu.sync_copy(src_hbm.at[wid, pl.ds(p * HC, HC)], sidx)
        pltpu.sync_copy(dst_hbm.at[wid, pl.ds(p * HC, HC)], didx)
        pltpu.async_copy(table_hbm.at[sidx.at[0]], rows.at[0], g0)
        pltpu.async_copy(table_hbm.at[sidx.at[1]], rows.at[1], g1)

        @pl.loop(0, HC, step=2)
        def _(j):
            pltpu.make_async_copy(table_hbm.at[sidx.at[j]], rows.at[0],
                                  g0).wait()
            pltpu.sync_copy(rows.at[0], acc_sh.at[didx.at[j]], add=True)

            @pl.when(j + 2 < HC)
            def _():
                pltpu.async_copy(table_hbm.at[sidx.at[j + 2]], rows.at[0], g0)

            pltpu.make_async_copy(table_hbm.at[sidx.at[j + 1]], rows.at[1],
                                  g1).wait()
            pltpu.sync_copy(rows.at[1], acc_sh.at[didx.at[j + 1]], add=True)

            @pl.when(j + 3 < HC)
            def _():
                pltpu.async_copy(table_hbm.at[sidx.at[j + 3]], rows.at[1], g1)

    plsc.subcore_barrier()
    pltpu.sync_copy(acc_sh.at[pl.ds(sid * RPS, RPS)],
                    acc_out.at[core, pl.ds(sid * RPS, RPS)])


@functools.cache
def _make_gather_kernel():
    return functools.partial(
        pl.kernel,
        out_type=jax.ShapeDtypeStruct((4 * B, D), jnp.float32),
        mesh=plsc.VectorSubcoreMesh(core_axis_name="core",
                                    subcore_axis_name="subcore"),
        scratch_types=[
            pltpu.VMEM((ICH, CHUNK), jnp.int32),
            pltpu.VMEM((CHUNK, D), jnp.float32),
        ],
    )(_gather_body)


def _gather_body(ids_hbm, zd_hbm, out_hbm, idx_v, rows_v):
    core = lax.axis_index("core")
    sid = lax.axis_index("subcore")
    wid = core * NS + sid
    pltpu.sync_copy(ids_hbm.at[wid], idx_v)

    @pl.loop(0, ICH)
    def _(j):
        pltpu.sync_copy(zd_hbm.at[idx_v.at[j]], rows_v)
        pltpu.sync_copy(rows_v, out_hbm.at[pl.ds(wid * IPW + j * CHUNK, CHUNK)])


_R = 1024
_G = NP // _R

_row_spec = pl.BlockSpec((_R, D), lambda i: (i, 0))
_col_spec = pl.BlockSpec((_R, 1), lambda i: (i, 0))
_acc_spec = pl.BlockSpec((NC, _R, D), lambda i: (0, i, 0))
_w_spec = pl.BlockSpec((D, D), lambda i: (0, 0))
_v_spec = pl.BlockSpec((1, D), lambda i: (0, 0))


def _pre_body(x_ref, deg_ref, dinv_ref, curs_ref):
    d = deg_ref[0, :, 0:1] + deg_ref[1, :, 0:1] + 1.0
    dv = lax.rsqrt(d)
    dinv_ref[...] = dv
    curs_ref[...] = x_ref[...] * dv


def _pre(xp, degp):
    return pl.pallas_call(
        _pre_body,
        grid=(_G,),
        in_specs=[_row_spec, pl.BlockSpec((NC, _R, DW), lambda i: (0, i, 0))],
        out_specs=[_col_spec, _row_spec],
        out_shape=[jax.ShapeDtypeStruct((NP, 1), jnp.float32),
                   jax.ShapeDtypeStruct((NP, D), jnp.float32)],
    )(xp, degp)


def _layer_body(acc_ref, cur_ref, dinv_ref, w_ref, b_ref, g_ref, bb_ref,
                cur_out, curs_out):
    dinv = dinv_ref[...]
    cur = cur_ref[...]
    agg = (acc_ref[0] + acc_ref[1] + cur * dinv) * dinv
    h = _dotT(agg, w_ref[...]) + b_ref[...]
    new = _ln(jnp.maximum(h, 0.0), g_ref[...], bb_ref[...])
    nxt = cur + new
    cur_out[...] = nxt
    curs_out[...] = nxt * dinv


def _layer(acc, cur, dinv, w, b, g, bb):
    return pl.pallas_call(
        _layer_body,
        grid=(_G,),
        in_specs=[_acc_spec, _row_spec, _col_spec, _w_spec, _v_spec, _v_spec,
                  _v_spec],
        out_specs=[_row_spec, _row_spec],
        out_shape=[jax.ShapeDtypeStruct((NP, D), jnp.float32),
                   jax.ShapeDtypeStruct((NP, D), jnp.float32)],
    )(acc, cur, dinv, w, b, g, bb)


def _final_body(acc_ref, e0_ref, e1_ref, dinv_ref, w2_ref, b2_ref,
                g_ref, bb_ref, wq_ref, wk_ref, wv_ref, wo_ref, bo_ref,
                zd_ref):
    dinv = dinv_ref[...]
    e1 = e1_ref[...]
    g = g_ref[...]
    bb = bb_ref[...]
    agg = (acc_ref[0] + acc_ref[1] + e1 * dinv) * dinv
    h = _dotT(agg, w2_ref[...]) + b2_ref[...]
    new = _ln(jnp.maximum(h, 0.0), g, bb)
    e2 = e1 + new
    e0 = e0_ref[...]

    q = _dotT(e2, wq_ref[...])
    ri = lax.broadcasted_iota(jnp.int32, (D, D), 0) // HD
    ci = lax.broadcasted_iota(jnp.int32, (D, D), 1) // HD
    rmat = jnp.where(ri == ci, 1.0, 0.0).astype(jnp.float32)

    def srep(e):
        k = _dotT(e, wk_ref[...])
        v = _dotT(e, wv_ref[...])
        s = lax.dot_general(q * k, rmat, (((1,), (0,)), ((), ())),
                            precision=_HIGH,
                            preferred_element_type=jnp.float32) * 0.25
        return s, v

    s0, v0 = srep(e0)
    s1, v1 = srep(e1)
    s2, v2 = srep(e2)
    m = jnp.maximum(jnp.maximum(s0, s1), s2)
    a0 = jnp.exp(s0 - m)
    a1 = jnp.exp(s1 - m)
    a2 = jnp.exp(s2 - m)
    ctx = (a0 * v0 + a1 * v1 + a2 * v2) / (a0 + a1 + a2)
    z = _ln(ctx, g, bb)
    zd_ref[...] = _dotT(z, wo_ref[...]) + bo_ref[...]


def _final(acc, e0, e1, dinv, w2, b2, g, bb, wq, wk, wv, wo, bo):
    return pl.pallas_call(
        _final_body,
        grid=(_G,),
        in_specs=[_acc_spec, _row_spec, _row_spec, _col_spec, _w_spec, _v_spec,
                  _v_spec, _v_spec, _w_spec, _w_spec, _w_spec, _w_spec,
                  _v_spec],
        out_specs=[_row_spec],
        out_shape=[jax.ShapeDtypeStruct((NP, D), jnp.float32)],
    )(acc, e0, e1, dinv, w2, b2, g, bb, wq, wk, wv, wo, bo)[0]


def kernel(x, edge_index, drug_pos_ids, target_pos_ids, drug_neg_ids,
           target_neg_ids, adjacency_matrix, W_gcn0, b_gcn0, W_gcn1, b_gcn1,
           W_gcn2, b_gcn2, W_Q, W_K, W_V, ln_g, ln_b, W_out, b_out):
    xp = jnp.pad(x, ((0, NP - N), (0, 0)))
    pad_idx = N + (jnp.arange(EPAD, dtype=jnp.int32) % (NP - N))
    src3 = jnp.concatenate([edge_index[0], pad_idx]).reshape(NW, NCHUNK, CHUNK)
    dst3 = jnp.concatenate([edge_index[1], pad_idx]).reshape(NW, NCHUNK, CHUNK)
    zeros_deg = jnp.zeros((RPS, DW), jnp.float32)
    zeros_rows = jnp.zeros((RPS, D), jnp.float32)
    ones_col = jnp.ones((CHUNK, DW), jnp.float32)

    b0 = b_gcn0.reshape(1, D)
    b1 = b_gcn1.reshape(1, D)
    b2 = b_gcn2.reshape(1, D)
    g = ln_g.reshape(1, D)
    bb = ln_b.reshape(1, D)
    bo = b_out.reshape(1, D)

    _deg_kernel = _make_deg_kernel()
    _edge_kernel = _make_edge_kernel()
    _gather_kernel = _make_gather_kernel()

    degp = _deg_kernel(dst3, zeros_deg, ones_col)
    dinv, curs0 = _pre(xp, degp)
    acc0 = _edge_kernel(src3, dst3, curs0, zeros_rows)
    cur1, curs1 = _layer(acc0, xp, dinv, W_gcn0, b0, g, bb)
    acc1 = _edge_kernel(src3, dst3, curs1, zeros_rows)
    cur2, curs2 = _layer(acc1, cur1, dinv, W_gcn1, b1, g, bb)
    acc2 = _edge_kernel(src3, dst3, curs2, zeros_rows)
    zd = _final(acc2, cur1, cur2, dinv, W_gcn2, b2, g, bb, W_Q, W_K, W_V,
                W_out, bo)

    ids = jnp.concatenate([drug_pos_ids, target_pos_ids, drug_neg_ids,
                           target_neg_ids]).reshape(NW, ICH, CHUNK)
    outg = _gather_kernel(ids, zd)
    return (outg[0:B], outg[B:2 * B], outg[2 * B:3 * B], outg[3 * B:4 * B])


# speedup vs baseline: 28.7621x; 1.2846x over previous
import dataclasses
import functools

import jax
import jax.numpy as jnp
from jax import lax
from jax.experimental import pallas as pl
from jax.experimental.pallas import tpu as pltpu
from jax.experimental.pallas import tpu_sc as plsc

N = 10000
NP = 10240
E = 320000
D = 128
HEADS = 8
HD = 16
B = 4096
EPS = 1e-5

NC = 2
NS = 16
NW = NC * NS
CHUNK = 128
NCHUNK = 80
HC = NCHUNK // 2
EPAD = NW * NCHUNK * CHUNK - E
RPS = NP // NS

IPW = 4 * B // NW
ICH = IPW // CHUNK

_PREC = lax.Precision.DEFAULT


def _dotT(a, w):
    return lax.dot_general(a, w, (((1,), (1,)), ((), ())),
                           precision=_PREC, preferred_element_type=jnp.float32)


def _ln(t, g, b):
    m = jnp.mean(t, axis=-1, keepdims=True)
    v = jnp.mean((t - m) ** 2, axis=-1, keepdims=True)
    return (t - m) * lax.rsqrt(v + EPS) * g + b


HR = NP // D
RPH = 8
NSH = HR // RPH


@functools.cache
def _make_deg_kernel():
    cp = pltpu.CompilerParams()
    if "needs_layout_passes" in pltpu.CompilerParams.__dataclass_fields__:
        cp = dataclasses.replace(cp, needs_layout_passes=False)
    return functools.partial(
        pl.kernel,
        out_type=jax.ShapeDtypeStruct((NC, HR, D), jnp.float32),
        mesh=plsc.VectorSubcoreMesh(core_axis_name="core",
                                    subcore_axis_name="subcore"),
        compiler_params=cp,
        scratch_types=[
            pltpu.VMEM((NCHUNK, CHUNK), jnp.int32),
            pltpu.VMEM((HR, D), jnp.float32),
            pltpu.VMEM((1, HR), jnp.int32),
            pltpu.VMEM_SHARED((HR, D), jnp.float32),
        ],
    )(_deg_body)


def _deg_body(dst_hbm, zeros_hbm, ident_hbm, deg_out, idx_v, hist_v, ident_v,
              deg_sh):
    core = lax.axis_index("core")
    sid = lax.axis_index("subcore")
    wid = core * NS + sid
    pltpu.sync_copy(zeros_hbm.at[pl.ds(0, HR)], hist_v)

    @pl.when(sid < NSH)
    def _():
        pltpu.sync_copy(zeros_hbm.at[pl.ds(0, RPH)],
                        deg_sh.at[pl.ds(sid * RPH, RPH)])

    pltpu.sync_copy(dst_hbm.at[wid], idx_v)
    pltpu.sync_copy(ident_hbm, ident_v)
    plsc.subcore_barrier()
    ones16 = jnp.ones((16,), jnp.float32)

    @pl.loop(0, NCHUNK)
    def _(j):
        for g in range(CHUNK // 16):
            v = idx_v[j, pl.ds(g * 16, 16)]
            row = lax.shift_right_logical(v, 7)
            col = lax.bitwise_and(v, 127)
            plsc.addupdate_scatter(hist_v, [row, col], ones16)

    pltpu.sync_copy(hist_v, deg_sh.at[ident_v.at[0]], add=True)
    plsc.subcore_barrier()

    @pl.when(sid < NSH)
    def _():
        pltpu.sync_copy(deg_sh.at[pl.ds(sid * RPH, RPH)],
                        deg_out.at[core, pl.ds(sid * RPH, RPH)])


@functools.cache
def _make_edge_kernel():
    return functools.partial(
        pl.kernel,
        out_type=jax.ShapeDtypeStruct((NC, NP, D), jnp.float32),
        mesh=plsc.VectorSubcoreMesh(core_axis_name="core",
                                    subcore_axis_name="subcore"),
        scratch_types=[
            pltpu.VMEM((HC, CHUNK), jnp.int32),
            pltpu.VMEM((HC, CHUNK), jnp.int32),
            pltpu.VMEM((2, CHUNK, D), jnp.float32),
            pltpu.VMEM_SHARED((NP, D), jnp.float32),
            pltpu.SemaphoreType.DMA,
            pltpu.SemaphoreType.DMA,
        ],
    )(_edge_body)


def _edge_body(src_hbm, dst_hbm, table_hbm, zeros_hbm, acc_out,
               sidx, didx, rows, acc_sh, g0, g1):
    core = lax.axis_index("core")
    sid = lax.axis_index("subcore")
    wid = core * NS + sid
    pltpu.sync_copy(zeros_hbm, acc_sh.at[pl.ds(sid * RPS, RPS)])
    plsc.subcore_barrier()

    @pl.loop(0, 2)
    def _(p):
        pltpu.sync_copy(src_hbm.at[wid, pl.ds(p * HC, HC)], sidx)
        pltpu.sync_copy(dst_hbm.at[wid, pl.ds(p * HC, HC)], didx)
        pltpu.async_copy(table_hbm.at[sidx.at[0]], rows.at[0], g0)
        pltpu.async_copy(table_hbm.at[sidx.at[1]], rows.at[1], g1)

        @pl.loop(0, HC, step=2)
        def _(j):
            pltpu.make_async_copy(table_hbm.at[sidx.at[j]], rows.at[0],
                                  g0).wait()
            pltpu.sync_copy(rows.at[0], acc_sh.at[didx.at[j]], add=True)

            @pl.when(j + 2 < HC)
            def _():
                pltpu.async_copy(table_hbm.at[sidx.at[j + 2]], rows.at[0], g0)

            pltpu.make_async_copy(table_hbm.at[sidx.at[j + 1]], rows.at[1],
                                  g1).wait()
            pltpu.sync_copy(rows.at[1], acc_sh.at[didx.at[j + 1]], add=True)

            @pl.when(j + 3 < HC)
            def _():
                pltpu.async_copy(table_hbm.at[sidx.at[j + 3]], rows.at[1], g1)

    plsc.subcore_barrier()
    pltpu.sync_copy(acc_sh.at[pl.ds(sid * RPS, RPS)],
                    acc_out.at[core, pl.ds(sid * RPS, RPS)])


WPQ = B // IPW


@functools.cache
def _make_gather_kernel():
    return functools.partial(
        pl.kernel,
        out_type=[jax.ShapeDtypeStruct((B, D), jnp.float32)] * 4,
        mesh=plsc.VectorSubcoreMesh(core_axis_name="core",
                                    subcore_axis_name="subcore"),
        scratch_types=[
            pltpu.VMEM((ICH, CHUNK), jnp.int32),
            pltpu.VMEM((CHUNK, D), jnp.float32),
        ],
    )(_gather_body)


def _gather_body(ids_hbm, zd_hbm, o0, o1, o2, o3, idx_v, rows_v):
    core = lax.axis_index("core")
    sid = lax.axis_index("subcore")
    wid = core * NS + sid
    pltpu.sync_copy(ids_hbm.at[wid], idx_v)

    @pl.loop(0, ICH)
    def _(j):
        pltpu.sync_copy(zd_hbm.at[idx_v.at[j]], rows_v)
        for q, o in enumerate((o0, o1, o2, o3)):
            @pl.when(wid // WPQ == q)
            def _(o=o, q=q):
                pltpu.sync_copy(
                    rows_v,
                    o.at[pl.ds((wid - q * WPQ) * IPW + j * CHUNK, CHUNK)])


_R = 2048
_G = NP // _R

_row_spec = pl.BlockSpec((_R, D), lambda i: (i, 0))
_col_spec = pl.BlockSpec((_R, 1), lambda i: (i, 0))
_acc_spec = pl.BlockSpec((NC, _R, D), lambda i: (0, i, 0))
_w_spec = pl.BlockSpec((D, D), lambda i: (0, 0))
_v_spec = pl.BlockSpec((1, D), lambda i: (0, 0))


def _pre_body(x_ref, deg_ref, dinv_ref, curs_ref):
    d = deg_ref[0] + deg_ref[1] + 1.0
    dvt = jnp.transpose(lax.rsqrt(d))
    for k in range(_R // D):
        col = dvt[:, k:k + 1]
        dinv_ref[pl.ds(k * D, D), :] = col
        curs_ref[pl.ds(k * D, D), :] = x_ref[pl.ds(k * D, D), :] * col


def _pre(xp, degp):
    return pl.pallas_call(
        _pre_body,
        grid=(_G,),
        in_specs=[_row_spec,
                  pl.BlockSpec((NC, _R // D, D), lambda i: (0, i, 0))],
        out_specs=[_col_spec, _row_spec],
        out_shape=[jax.ShapeDtypeStruct((NP, 1), jnp.float32),
                   jax.ShapeDtypeStruct((NP, D), jnp.float32)],
    )(xp, degp)


def _layer_body(acc_ref, cur_ref, dinv_ref, w_ref, b_ref, g_ref, bb_ref,
                cur_out, curs_out):
    dinv = dinv_ref[...]
    cur = cur_ref[...]
    agg = (acc_ref[0] + acc_ref[1] + cur * dinv) * dinv
    h = _dotT(agg, w_ref[...]) + b_ref[...]
    new = _ln(jnp.maximum(h, 0.0), g_ref[...], bb_ref[...])
    nxt = cur + new
    cur_out[...] = nxt
    curs_out[...] = nxt * dinv


def _layer(acc, cur, dinv, w, b, g, bb):
    return pl.pallas_call(
        _layer_body,
        grid=(_G,),
        in_specs=[_acc_spec, _row_spec, _col_spec, _w_spec, _v_spec, _v_spec,
                  _v_spec],
        out_specs=[_row_spec, _row_spec],
        out_shape=[jax.ShapeDtypeStruct((NP, D), jnp.float32),
                   jax.ShapeDtypeStruct((NP, D), jnp.float32)],
    )(acc, cur, dinv, w, b, g, bb)


def _final_body(acc_ref, e0_ref, e1_ref, dinv_ref, w2_ref, b2_ref,
                g_ref, bb_ref, wq_ref, wk_ref, wv_ref, wo_ref, bo_ref,
                zd_ref):
    dinv = dinv_ref[...]
    e1 = e1_ref[...]
    g = g_ref[...]
    bb = bb_ref[...]
    agg = (acc_ref[0] + acc_ref[1] + e1 * dinv) * dinv
    h = _dotT(agg, w2_ref[...]) + b2_ref[...]
    new = _ln(jnp.maximum(h, 0.0), g, bb)
    e2 = e1 + new
    e0 = e0_ref[...]

    q = _dotT(e2, wq_ref[...])
    ri = lax.broadcasted_iota(jnp.int32, (D, D), 0) // HD
    ci = lax.broadcasted_iota(jnp.int32, (D, D), 1) // HD
    rmat = jnp.where(ri == ci, 1.0, 0.0).astype(jnp.float32)

    def srep(e):
        k = _dotT(e, wk_ref[...])
        v = _dotT(e, wv_ref[...])
        s = lax.dot_general(q * k, rmat, (((1,), (0,)), ((), ())),
                            precision=_PREC,
                            preferred_element_type=jnp.float32) * 0.25
        return s, v

    s0, v0 = srep(e0)
    s1, v1 = srep(e1)
    s2, v2 = srep(e2)
    m = jnp.maximum(jnp.maximum(s0, s1), s2)
    a0 = jnp.exp(s0 - m)
    a1 = jnp.exp(s1 - m)
    a2 = jnp.exp(s2 - m)
    ctx = (a0 * v0 + a1 * v1 + a2 * v2) / (a0 + a1 + a2)
    z = _ln(ctx, g, bb)
    zd_ref[...] = _dotT(z, wo_ref[...]) + bo_ref[...]


def _final(acc, e0, e1, dinv, w2, b2, g, bb, wq, wk, wv, wo, bo):
    return pl.pallas_call(
        _final_body,
        grid=(_G,),
        in_specs=[_acc_spec, _row_spec, _row_spec, _col_spec, _w_spec, _v_spec,
                  _v_spec, _v_spec, _w_spec, _w_spec, _w_spec, _w_spec,
                  _v_spec],
        out_specs=[_row_spec],
        out_shape=[jax.ShapeDtypeStruct((NP, D), jnp.float32)],
    )(acc, e0, e1, dinv, w2, b2, g, bb, wq, wk, wv, wo, bo)[0]


def kernel(x, edge_index, drug_pos_ids, target_pos_ids, drug_neg_ids,
           target_neg_ids, adjacency_matrix, W_gcn0, b_gcn0, W_gcn1, b_gcn1,
           W_gcn2, b_gcn2, W_Q, W_K, W_V, ln_g, ln_b, W_out, b_out):
    xp = jnp.pad(x, ((0, NP - N), (0, 0)))
    pad_idx = N + (jnp.arange(EPAD, dtype=jnp.int32) % (NP - N))
    src3 = jnp.concatenate([edge_index[0], pad_idx]).reshape(NW, NCHUNK, CHUNK)
    dst3 = jnp.concatenate([edge_index[1], pad_idx]).reshape(NW, NCHUNK, CHUNK)
    zeros_rows = jnp.zeros((RPS, D), jnp.float32)
    ident = jnp.arange(HR, dtype=jnp.int32).reshape(1, HR)

    b0 = b_gcn0.reshape(1, D)
    b1 = b_gcn1.reshape(1, D)
    b2 = b_gcn2.reshape(1, D)
    g = ln_g.reshape(1, D)
    bb = ln_b.reshape(1, D)
    bo = b_out.reshape(1, D)

    _deg_kernel = _make_deg_kernel()
    _edge_kernel = _make_edge_kernel()
    _gather_kernel = _make_gather_kernel()

    degp = _deg_kernel(dst3, zeros_rows, ident)
    dinv, curs0 = _pre(xp, degp)
    acc0 = _edge_kernel(src3, dst3, curs0, zeros_rows)
    cur1, curs1 = _layer(acc0, xp, dinv, W_gcn0, b0, g, bb)
    acc1 = _edge_kernel(src3, dst3, curs1, zeros_rows)
    cur2, curs2 = _layer(acc1, cur1, dinv, W_gcn1, b1, g, bb)
    acc2 = _edge_kernel(src3, dst3, curs2, zeros_rows)
    zd = _final(acc2, cur1, cur2, dinv, W_gcn2, b2, g, bb, W_Q, W_K, W_V,
                W_out, bo)

    ids = jnp.concatenate([drug_pos_ids, target_pos_ids, drug_neg_ids,
                           target_neg_ids]).reshape(NW, ICH, CHUNK)
    return tuple(_gather_kernel(ids, zd))
